# Initial kernel scaffold; baseline (speedup 1.0000x reference)
import jax
import jax.numpy as jnp
from jax.experimental import pallas as pl


def kernel(x, edge_index, edge_weight, W0, b0, W1, b1, W2, b2):
    raise NotImplementedError("write your pallas kernel here")


# speedup vs baseline: 49.1645x; 49.1645x over previous
import functools

import jax
import jax.numpy as jnp
from jax import lax
from jax.experimental import pallas as pl
from jax.experimental.pallas import tpu as pltpu
from jax.experimental.pallas import tpu_sc as plsc

_N = 10000
_E = 160000
_NC = 2
_NS = 16
_NW = _NC * _NS
_L = 16
_C = 5008
_EPAD = _NW * _C


def _prep_body(x_ref, w0_ref, w1_ref, w2_ref, b0_ref, b1_ref,
               u0_ref, c0_ref, c1_ref):
    hi = lax.Precision.HIGHEST
    w12 = jnp.dot(w1_ref[...], w2_ref[...], precision=hi)
    w012 = jnp.dot(w0_ref[...], w12, precision=hi)
    u0_ref[...] = jnp.dot(x_ref[...], w012, precision=hi)
    c0_ref[...] = jnp.dot(b0_ref[...][None, :], w12, precision=hi)
    c1_ref[...] = jnp.dot(b1_ref[...][None, :], w2_ref[...], precision=hi)


_prep = pl.pallas_call(
    _prep_body,
    out_shape=(
        jax.ShapeDtypeStruct((_N, 1), jnp.float32),
        jax.ShapeDtypeStruct((1, 1), jnp.float32),
        jax.ShapeDtypeStruct((1, 1), jnp.float32),
    ),
)


def _reduce_body(acc_ref, beta_ref, out_ref):
    out_ref[...] = jnp.sum(acc_ref[...], axis=0, keepdims=True) + beta_ref[0, 0]


_reduce = pl.pallas_call(
    _reduce_body,
    in_specs=[
        pl.BlockSpec(memory_space=pltpu.VMEM),
        pl.BlockSpec(memory_space=pltpu.SMEM),
    ],
    out_shape=jax.ShapeDtypeStruct((1, _N), jnp.float32),
)


_mesh = plsc.VectorSubcoreMesh(
    core_axis_name="c", subcore_axis_name="s",
    num_cores=_NC, num_subcores=_NS)


@functools.partial(
    pl.kernel,
    out_type=jax.ShapeDtypeStruct((_NW, _N), jnp.float32),
    mesh=_mesh,
    compiler_params=pltpu.CompilerParams(needs_layout_passes=False),
    scratch_types=[
        pltpu.VMEM((_N,), jnp.float32),
        pltpu.VMEM((_N,), jnp.float32),
        pltpu.VMEM((_C,), jnp.int32),
        pltpu.VMEM((_C,), jnp.int32),
        pltpu.VMEM((_C,), jnp.float32),
    ],
)
def _sc_pass(u_hbm, src_hbm, dst_hbm, w_hbm, out_hbm,
             u_v, acc_v, src_v, dst_v, w_v):
    cid = lax.axis_index("c")
    sid = lax.axis_index("s")
    wid = sid * _NC + cid
    base = wid * _C

    pltpu.sync_copy(u_hbm, u_v)
    pltpu.sync_copy(src_hbm.at[pl.ds(base, _C)], src_v)
    pltpu.sync_copy(dst_hbm.at[pl.ds(base, _C)], dst_v)
    pltpu.sync_copy(w_hbm.at[pl.ds(base, _C)], w_v)

    def _zero(i, carry):
        acc_v[pl.ds(i * _L, _L)] = jnp.zeros((_L,), jnp.float32)
        return carry

    lax.fori_loop(0, _N // _L, _zero, 0)

    def _edges(i, carry):
        s = src_v[pl.ds(i * _L, _L)]
        d = dst_v[pl.ds(i * _L, _L)]
        wv = w_v[pl.ds(i * _L, _L)]
        vals = plsc.load_gather(u_v, [s]) * wv
        plsc.addupdate_scatter(acc_v, [d], vals)
        return carry

    lax.fori_loop(0, _C // _L, _edges, 0)

    pltpu.sync_copy(acc_v, out_hbm.at[wid])


def kernel(x, edge_index, edge_weight, W0, b0, W1, b1, W2, b2):
    src = edge_index[0]
    dst = edge_index[1]
    pad = _EPAD - _E
    src = jnp.concatenate([src, jnp.zeros((pad,), jnp.int32)])
    dst = jnp.concatenate([dst, jnp.zeros((pad,), jnp.int32)])
    w = jnp.concatenate([edge_weight, jnp.zeros((pad,), jnp.float32)])

    u0, c0, c1 = _prep(x, W0, W1, W2, b0, b1)

    u = u0.reshape(_N)
    acc = _sc_pass(u, src, dst, w)
    u = _reduce(acc, c0).reshape(_N)
    acc = _sc_pass(u, src, dst, w)
    u = _reduce(acc, c1).reshape(_N)
    acc = _sc_pass(u, src, dst, w)
    y = _reduce(acc, b2.reshape(1, 1)).reshape(_N, 1)
    return y


# speedup vs baseline: 52.6196x; 1.0703x over previous
import functools

import jax
import jax.numpy as jnp
from jax import lax
from jax.experimental import pallas as pl
from jax.experimental.pallas import tpu as pltpu
from jax.experimental.pallas import tpu_sc as plsc

_N = 10000
_E = 160000
_NC = 2
_NS = 16
_NW = _NC * _NS
_L = 16
_C = 5008
_EPAD = _NW * _C


def _prep_body(x_ref, w0_ref, w1_ref, w2_ref, b0_ref, b1_ref,
               u0_ref, c0_ref, c1_ref):
    hi = lax.Precision.HIGHEST
    w12 = jnp.dot(w1_ref[...], w2_ref[...], precision=hi)
    w012 = jnp.dot(w0_ref[...], w12, precision=hi)
    u0_ref[...] = jnp.dot(x_ref[...], w012, precision=hi)
    c0_ref[...] = jnp.dot(b0_ref[...][None, :], w12, precision=hi)
    c1_ref[...] = jnp.dot(b1_ref[...][None, :], w2_ref[...], precision=hi)


_prep = pl.pallas_call(
    _prep_body,
    out_shape=(
        jax.ShapeDtypeStruct((_N, 1), jnp.float32),
        jax.ShapeDtypeStruct((1, 1), jnp.float32),
        jax.ShapeDtypeStruct((1, 1), jnp.float32),
    ),
)


def _reduce_body(acc_ref, beta_ref, out_ref):
    out_ref[...] = jnp.sum(acc_ref[...], axis=0, keepdims=True) + beta_ref[0, 0]


_reduce = pl.pallas_call(
    _reduce_body,
    in_specs=[
        pl.BlockSpec(memory_space=pltpu.VMEM),
        pl.BlockSpec(memory_space=pltpu.SMEM),
    ],
    out_shape=jax.ShapeDtypeStruct((1, _N), jnp.float32),
)


_mesh = plsc.VectorSubcoreMesh(
    core_axis_name="c", subcore_axis_name="s",
    num_cores=_NC, num_subcores=_NS)


@functools.partial(
    pl.kernel,
    out_type=jax.ShapeDtypeStruct((_NW, _N), jnp.float32),
    mesh=_mesh,
    compiler_params=pltpu.CompilerParams(needs_layout_passes=False),
    scratch_types=[
        pltpu.VMEM((_N,), jnp.float32),
        pltpu.VMEM((_N,), jnp.float32),
        pltpu.VMEM((_C,), jnp.int32),
        pltpu.VMEM((_C,), jnp.int32),
        pltpu.VMEM((_C,), jnp.float32),
    ],
)
def _sc_pass(u_hbm, src_hbm, dst_hbm, w_hbm, z_hbm, out_hbm,
             u_v, acc_v, src_v, dst_v, w_v):
    cid = lax.axis_index("c")
    sid = lax.axis_index("s")
    wid = sid * _NC + cid
    base = wid * _C

    pltpu.sync_copy(u_hbm, u_v)
    pltpu.sync_copy(src_hbm.at[pl.ds(base, _C)], src_v)
    pltpu.sync_copy(dst_hbm.at[pl.ds(base, _C)], dst_v)
    pltpu.sync_copy(w_hbm.at[pl.ds(base, _C)], w_v)
    pltpu.sync_copy(z_hbm, acc_v)

    @plsc.parallel_loop(0, _C // _L, unroll=8)
    def _edges(i):
        s = src_v[pl.ds(i * _L, _L)]
        d = dst_v[pl.ds(i * _L, _L)]
        wv = w_v[pl.ds(i * _L, _L)]
        vals = plsc.load_gather(u_v, [s]) * wv
        plsc.addupdate_scatter(acc_v, [d], vals)

    pltpu.sync_copy(acc_v, out_hbm.at[wid])


def kernel(x, edge_index, edge_weight, W0, b0, W1, b1, W2, b2):
    src = edge_index[0]
    dst = edge_index[1]
    pad = _EPAD - _E
    src = jnp.concatenate([src, jnp.zeros((pad,), jnp.int32)])
    dst = jnp.concatenate([dst, jnp.zeros((pad,), jnp.int32)])
    w = jnp.concatenate([edge_weight, jnp.zeros((pad,), jnp.float32)])

    u0, c0, c1 = _prep(x, W0, W1, W2, b0, b1)

    z = jnp.zeros((_N,), jnp.float32)
    u = u0.reshape(_N)
    acc = _sc_pass(u, src, dst, w, z)
    u = _reduce(acc, c0).reshape(_N)
    acc = _sc_pass(u, src, dst, w, z)
    u = _reduce(acc, c1).reshape(_N)
    acc = _sc_pass(u, src, dst, w, z)
    y = _reduce(acc, b2.reshape(1, 1)).reshape(_N, 1)
    return y


# speedup vs baseline: 57.9647x; 1.1016x over previous
import functools

import jax
import jax.numpy as jnp
from jax import lax
from jax.experimental import pallas as pl
from jax.experimental.pallas import tpu as pltpu
from jax.experimental.pallas import tpu_sc as plsc

_N = 10000
_NP = 10240
_SL = 640
_E = 160000
_NS = 16
_C = _E // _NS
_L = 16


def _prep_body(x_ref, w0_ref, w1_ref, w2_ref, b0_ref, b1_ref,
               u0_ref, c0_ref, c1_ref):
    hi = lax.Precision.HIGHEST
    w12 = jnp.dot(w1_ref[...], w2_ref[...], precision=hi)
    w012 = jnp.dot(w0_ref[...], w12, precision=hi)
    u0_ref[...] = jnp.dot(x_ref[...], w012, precision=hi)
    c0_ref[...] = jnp.dot(b0_ref[...][None, :], w12, precision=hi)
    c1_ref[...] = jnp.dot(b1_ref[...][None, :], w2_ref[...], precision=hi)


_prep = pl.pallas_call(
    _prep_body,
    out_shape=(
        jax.ShapeDtypeStruct((_N, 1), jnp.float32),
        jax.ShapeDtypeStruct((1, 1), jnp.float32),
        jax.ShapeDtypeStruct((1, 1), jnp.float32),
    ),
)


_mesh = plsc.VectorSubcoreMesh(
    core_axis_name="c", subcore_axis_name="s",
    num_cores=1, num_subcores=_NS)


@functools.partial(
    pl.kernel,
    out_type=(
        jax.ShapeDtypeStruct((_NP,), jnp.float32),
        jax.ShapeDtypeStruct((_NS, _NP), jnp.float32),
        jax.ShapeDtypeStruct((_NP,), jnp.float32),
    ),
    mesh=_mesh,
    compiler_params=pltpu.CompilerParams(needs_layout_passes=False),
    scratch_types=[
        pltpu.VMEM((_NP,), jnp.float32),
        pltpu.VMEM((_NP,), jnp.float32),
        pltpu.VMEM((_C,), jnp.int32),
        pltpu.VMEM((_C,), jnp.int32),
        pltpu.VMEM((_C,), jnp.float32),
        pltpu.VMEM((_NS, _SL), jnp.float32),
        pltpu.VMEM((_SL,), jnp.float32),
        pltpu.VMEM((3, _L), jnp.float32),
    ],
)
def _sc_fused(u_hbm, src_hbm, dst_hbm, w_hbm, z_hbm, betas_hbm,
              y_hbm, p_hbm, ux_hbm,
              u_v, acc_v, src_v, dst_v, w_v, red_v, sl_v, betas_v):
    sid = lax.axis_index("s")
    base = sid * _C
    nbase = sid * _SL

    pltpu.sync_copy(u_hbm, u_v)
    pltpu.sync_copy(src_hbm.at[pl.ds(base, _C)], src_v)
    pltpu.sync_copy(dst_hbm.at[pl.ds(base, _C)], dst_v)
    pltpu.sync_copy(w_hbm.at[pl.ds(base, _C)], w_v)
    pltpu.sync_copy(betas_hbm, betas_v)

    for p in range(3):
        pltpu.sync_copy(z_hbm, acc_v)

        @plsc.parallel_loop(0, _C // _L, unroll=8)
        def _edges(i):
            s = src_v[pl.ds(i * _L, _L)]
            d = dst_v[pl.ds(i * _L, _L)]
            wv = w_v[pl.ds(i * _L, _L)]
            vals = plsc.load_gather(u_v, [s]) * wv
            plsc.addupdate_scatter(acc_v, [d], vals)

        pltpu.sync_copy(acc_v, p_hbm.at[sid])
        plsc.subcore_barrier()

        pltpu.sync_copy(p_hbm.at[:, pl.ds(nbase, _SL)], red_v)
        bvec = betas_v[p, :]
        for k in range(_SL // _L):
            v = red_v[0, pl.ds(k * _L, _L)]
            for j in range(1, _NS):
                v = v + red_v[j, pl.ds(k * _L, _L)]
            sl_v[pl.ds(k * _L, _L)] = v + bvec

        if p < 2:
            pltpu.sync_copy(sl_v, ux_hbm.at[pl.ds(nbase, _SL)])
            plsc.subcore_barrier()
            pltpu.sync_copy(ux_hbm, u_v)
        else:
            pltpu.sync_copy(sl_v, y_hbm.at[pl.ds(nbase, _SL)])


def kernel(x, edge_index, edge_weight, W0, b0, W1, b1, W2, b2):
    src = edge_index[0]
    dst = edge_index[1]

    u0, c0, c1 = _prep(x, W0, W1, W2, b0, b1)

    u = jnp.concatenate([u0.reshape(_N), jnp.zeros((_NP - _N,), jnp.float32)])
    z = jnp.zeros((_NP,), jnp.float32)
    betas = jnp.concatenate([
        jnp.broadcast_to(c0, (1, _L)),
        jnp.broadcast_to(c1, (1, _L)),
        jnp.broadcast_to(b2[None, :], (1, _L)),
    ])

    y_pad, _, _ = _sc_fused(u, src, dst, edge_weight, z, betas)
    return y_pad[:_N].reshape(_N, 1)


# speedup vs baseline: 60.4354x; 1.0426x over previous
import functools

import jax
import jax.numpy as jnp
from jax import lax
from jax.experimental import pallas as pl
from jax.experimental.pallas import tpu as pltpu
from jax.experimental.pallas import tpu_sc as plsc

_N = 10000
_NP = 10240
_SL = 640
_E = 160000
_NS = 16
_C = _E // _NS
_L = 16


def _prep_body(x_ref, w0_ref, w1_ref, w2_ref, b0_ref, b1_ref, b2_ref,
               u0_ref, betas_ref):
    hi = lax.Precision.HIGHEST
    w12 = jnp.dot(w1_ref[...], w2_ref[...], precision=hi)
    w012 = jnp.dot(w0_ref[...], w12, precision=hi)
    u0 = jnp.dot(x_ref[...], w012, precision=hi)
    u0_ref[...] = jnp.concatenate(
        [u0, jnp.zeros((_NP - _N, 1), jnp.float32)], axis=0)
    c0 = jnp.dot(b0_ref[...][None, :], w12, precision=hi)
    c1 = jnp.dot(b1_ref[...][None, :], w2_ref[...], precision=hi)
    betas_ref[...] = jnp.concatenate([
        jnp.broadcast_to(c0, (1, _L)),
        jnp.broadcast_to(c1, (1, _L)),
        jnp.broadcast_to(b2_ref[...][None, :], (1, _L)),
    ], axis=0)


_prep = pl.pallas_call(
    _prep_body,
    out_shape=(
        jax.ShapeDtypeStruct((_NP, 1), jnp.float32),
        jax.ShapeDtypeStruct((3, _L), jnp.float32),
    ),
)


_mesh = plsc.VectorSubcoreMesh(
    core_axis_name="c", subcore_axis_name="s",
    num_cores=1, num_subcores=_NS)


@functools.partial(
    pl.kernel,
    out_type=(
        jax.ShapeDtypeStruct((_N,), jnp.float32),
        jax.ShapeDtypeStruct((_NS, _NP), jnp.float32),
        jax.ShapeDtypeStruct((_NP,), jnp.float32),
    ),
    mesh=_mesh,
    compiler_params=pltpu.CompilerParams(needs_layout_passes=False),
    scratch_types=[
        pltpu.VMEM((_NP,), jnp.float32),
        pltpu.VMEM((_NP,), jnp.float32),
        pltpu.VMEM((_C,), jnp.int32),
        pltpu.VMEM((_C,), jnp.int32),
        pltpu.VMEM((_C,), jnp.float32),
        pltpu.VMEM((_NS, _SL), jnp.float32),
        pltpu.VMEM((_SL,), jnp.float32),
        pltpu.VMEM((3, _L), jnp.float32),
    ],
)
def _sc_fused(u_hbm, src_hbm, dst_hbm, w_hbm, z_hbm, betas_hbm,
              y_hbm, p_hbm, ux_hbm,
              u_v, acc_v, src_v, dst_v, w_v, red_v, sl_v, betas_v):
    sid = lax.axis_index("s")
    base = sid * _C
    nbase = sid * _SL

    pltpu.sync_copy(u_hbm, u_v)
    pltpu.sync_copy(src_hbm.at[pl.ds(base, _C)], src_v)
    pltpu.sync_copy(dst_hbm.at[pl.ds(base, _C)], dst_v)
    pltpu.sync_copy(w_hbm.at[pl.ds(base, _C)], w_v)
    pltpu.sync_copy(betas_hbm, betas_v)

    for p in range(3):
        pltpu.sync_copy(z_hbm, acc_v)

        @plsc.parallel_loop(0, _C // _L, unroll=8)
        def _edges(i):
            s = src_v[pl.ds(i * _L, _L)]
            d = dst_v[pl.ds(i * _L, _L)]
            wv = w_v[pl.ds(i * _L, _L)]
            vals = plsc.load_gather(u_v, [s]) * wv
            plsc.addupdate_scatter(acc_v, [d], vals)

        pltpu.sync_copy(acc_v, p_hbm.at[sid])
        plsc.subcore_barrier()

        pltpu.sync_copy(p_hbm.at[:, pl.ds(nbase, _SL)], red_v)
        bvec = betas_v[p, :]
        for k in range(_SL // _L):
            v = red_v[0, pl.ds(k * _L, _L)]
            for j in range(1, _NS):
                v = v + red_v[j, pl.ds(k * _L, _L)]
            sl_v[pl.ds(k * _L, _L)] = v + bvec

        if p < 2:
            pltpu.sync_copy(sl_v, ux_hbm.at[pl.ds(nbase, _SL)])
            plsc.subcore_barrier()
            pltpu.sync_copy(ux_hbm, u_v)
        else:
            @pl.when(sid < _NS - 1)
            def _():
                pltpu.sync_copy(sl_v, y_hbm.at[pl.ds(nbase, _SL)])

            @pl.when(sid == _NS - 1)
            def _():
                pltpu.sync_copy(sl_v.at[pl.ds(0, _N - (_NS - 1) * _SL)],
                                y_hbm.at[pl.ds(nbase, _N - (_NS - 1) * _SL)])


def kernel(x, edge_index, edge_weight, W0, b0, W1, b1, W2, b2):
    src = edge_index[0]
    dst = edge_index[1]

    u0, betas = _prep(x, W0, W1, W2, b0, b1, b2)

    z = jnp.zeros((_NP,), jnp.float32)
    y, _, _ = _sc_fused(u0.reshape(_NP), src, dst, edge_weight, z, betas)
    return y.reshape(_N, 1)


# speedup vs baseline: 74.4890x; 1.2325x over previous
import functools

import jax
import jax.numpy as jnp
from jax import lax
from jax.experimental import pallas as pl
from jax.experimental.pallas import tpu as pltpu
from jax.experimental.pallas import tpu_sc as plsc

_N = 10000
_NP = 10240
_SL = 640
_E = 160000
_NS = 16
_C = _E // _NS
_L = 16


def _prep_body(x_ref, ei_ref, w0_ref, w1_ref, w2_ref, b0_ref, b1_ref, b2_ref,
               u0_ref, betas_ref, src_ref, dst_ref):
    hi = lax.Precision.HIGHEST
    w12 = jnp.dot(w1_ref[...], w2_ref[...], precision=hi)
    w012 = jnp.dot(w0_ref[...], w12, precision=hi)
    xv = x_ref[...]
    x_hi = xv.astype(jnp.bfloat16)
    x_lo = (xv - x_hi.astype(jnp.float32)).astype(jnp.bfloat16)
    w_hi = w012.astype(jnp.bfloat16)
    w_lo = (w012 - w_hi.astype(jnp.float32)).astype(jnp.bfloat16)
    f32 = jnp.float32
    u0 = (jnp.dot(x_hi, w_hi, preferred_element_type=f32)
          + jnp.dot(x_lo, w_hi, preferred_element_type=f32)
          + jnp.dot(x_hi, w_lo, preferred_element_type=f32))
    u0_ref[...] = jnp.concatenate(
        [u0[:, 0], jnp.zeros((_NP - _N,), jnp.float32)], axis=0)
    c0 = jnp.dot(b0_ref[...][None, :], w12, precision=hi)
    c1 = jnp.dot(b1_ref[...][None, :], w2_ref[...], precision=hi)
    betas_ref[...] = jnp.concatenate([
        jnp.broadcast_to(c0, (1, _L)),
        jnp.broadcast_to(c1, (1, _L)),
        jnp.broadcast_to(b2_ref[...][None, :], (1, _L)),
    ], axis=0)
    ei = ei_ref[...]
    src_ref[...] = ei[0]
    dst_ref[...] = ei[1]


_prep = pl.pallas_call(
    _prep_body,
    out_shape=(
        jax.ShapeDtypeStruct((_NP,), jnp.float32),
        jax.ShapeDtypeStruct((3, _L), jnp.float32),
        jax.ShapeDtypeStruct((_E,), jnp.int32),
        jax.ShapeDtypeStruct((_E,), jnp.int32),
    ),
)


_mesh = plsc.VectorSubcoreMesh(
    core_axis_name="c", subcore_axis_name="s",
    num_cores=1, num_subcores=_NS)


@functools.partial(
    pl.kernel,
    out_type=(
        jax.ShapeDtypeStruct((_N,), jnp.float32),
        jax.ShapeDtypeStruct((_NS, _NP), jnp.float32),
        jax.ShapeDtypeStruct((_NP,), jnp.float32),
    ),
    mesh=_mesh,
    compiler_params=pltpu.CompilerParams(needs_layout_passes=False),
    scratch_types=[
        pltpu.VMEM((_NP,), jnp.float32),
        pltpu.VMEM((_NP,), jnp.float32),
        pltpu.VMEM((_C,), jnp.int32),
        pltpu.VMEM((_C,), jnp.int32),
        pltpu.VMEM((_C,), jnp.float32),
        pltpu.VMEM((_NS, _SL), jnp.float32),
        pltpu.VMEM((_SL,), jnp.float32),
        pltpu.VMEM((3, _L), jnp.float32),
    ],
)
def _sc_fused(u_hbm, src_hbm, dst_hbm, w_hbm, z_hbm, betas_hbm,
              y_hbm, p_hbm, ux_hbm,
              u_v, acc_v, src_v, dst_v, w_v, red_v, sl_v, betas_v):
    sid = lax.axis_index("s")
    base = sid * _C
    nbase = sid * _SL

    pltpu.sync_copy(u_hbm, u_v)
    pltpu.sync_copy(src_hbm.at[pl.ds(base, _C)], src_v)
    pltpu.sync_copy(dst_hbm.at[pl.ds(base, _C)], dst_v)
    pltpu.sync_copy(w_hbm.at[pl.ds(base, _C)], w_v)
    pltpu.sync_copy(betas_hbm, betas_v)

    for p in range(3):
        pltpu.sync_copy(z_hbm, acc_v)

        @plsc.parallel_loop(0, _C // _L, unroll=8)
        def _edges(i):
            s = src_v[pl.ds(i * _L, _L)]
            d = dst_v[pl.ds(i * _L, _L)]
            wv = w_v[pl.ds(i * _L, _L)]
            vals = plsc.load_gather(u_v, [s]) * wv
            plsc.addupdate_scatter(acc_v, [d], vals)

        pltpu.sync_copy(acc_v, p_hbm.at[sid])
        plsc.subcore_barrier()

        pltpu.sync_copy(p_hbm.at[:, pl.ds(nbase, _SL)], red_v)
        bvec = betas_v[p, :]
        for k in range(_SL // _L):
            v = red_v[0, pl.ds(k * _L, _L)]
            for j in range(1, _NS):
                v = v + red_v[j, pl.ds(k * _L, _L)]
            sl_v[pl.ds(k * _L, _L)] = v + bvec

        if p < 2:
            pltpu.sync_copy(sl_v, ux_hbm.at[pl.ds(nbase, _SL)])
            plsc.subcore_barrier()
            pltpu.sync_copy(ux_hbm, u_v)
        else:
            @pl.when(sid < _NS - 1)
            def _():
                pltpu.sync_copy(sl_v, y_hbm.at[pl.ds(nbase, _SL)])

            @pl.when(sid == _NS - 1)
            def _():
                pltpu.sync_copy(sl_v.at[pl.ds(0, _N - (_NS - 1) * _SL)],
                                y_hbm.at[pl.ds(nbase, _N - (_NS - 1) * _SL)])


def kernel(x, edge_index, edge_weight, W0, b0, W1, b1, W2, b2):
    u0, betas, src, dst = _prep(x, edge_index, W0, W1, W2, b0, b1, b2)

    z = jnp.zeros((_NP,), jnp.float32)
    y, _, _ = _sc_fused(u0, src, dst, edge_weight, z, betas)
    return y.reshape(_N, 1)


# speedup vs baseline: 83.0734x; 1.1152x over previous
import functools

import jax
import jax.numpy as jnp
from jax import lax
from jax.experimental import pallas as pl
from jax.experimental.pallas import tpu as pltpu
from jax.experimental.pallas import tpu_sc as plsc

_N = 10000
_NP = 10240
_SL = 640
_E = 160000
_NS = 16
_C = _E // _NS
_L = 16


def _prep_body(x_ref, ei_ref, w0_ref, w1_ref, w2_ref, b0_ref, b1_ref, b2_ref,
               u0_ref, betas_ref, src_ref, dst_ref):
    hi = lax.Precision.HIGHEST
    w12 = jnp.dot(w1_ref[...], w2_ref[...], precision=hi)
    w012 = jnp.dot(w0_ref[...], w12, precision=hi)
    xv = x_ref[...]
    x_hi = xv.astype(jnp.bfloat16)
    x_lo = (xv - x_hi.astype(jnp.float32)).astype(jnp.bfloat16)
    w_hi = w012.astype(jnp.bfloat16)
    w_lo = (w012 - w_hi.astype(jnp.float32)).astype(jnp.bfloat16)
    f32 = jnp.float32
    u0 = (jnp.dot(x_hi, w_hi, preferred_element_type=f32)
          + jnp.dot(x_lo, w_hi, preferred_element_type=f32)
          + jnp.dot(x_hi, w_lo, preferred_element_type=f32))
    u0_ref[...] = jnp.concatenate(
        [u0[:, 0], jnp.zeros((_NP - _N,), jnp.float32)], axis=0)
    c0 = jnp.dot(b0_ref[...][None, :], w12, precision=hi)
    c1 = jnp.dot(b1_ref[...][None, :], w2_ref[...], precision=hi)
    betas_ref[...] = jnp.concatenate([
        jnp.broadcast_to(c0, (1, _L)),
        jnp.broadcast_to(c1, (1, _L)),
        jnp.broadcast_to(b2_ref[...][None, :], (1, _L)),
    ], axis=0)
    ei = ei_ref[...]
    src_ref[...] = ei[0]
    dst_ref[...] = ei[1]


_prep = pl.pallas_call(
    _prep_body,
    out_shape=(
        jax.ShapeDtypeStruct((_NP,), jnp.float32),
        jax.ShapeDtypeStruct((3, _L), jnp.float32),
        jax.ShapeDtypeStruct((_E,), jnp.int32),
        jax.ShapeDtypeStruct((_E,), jnp.int32),
    ),
)


_mesh = plsc.VectorSubcoreMesh(
    core_axis_name="c", subcore_axis_name="s",
    num_cores=1, num_subcores=_NS)


@functools.partial(
    pl.kernel,
    out_type=(
        jax.ShapeDtypeStruct((_N,), jnp.float32),
        jax.ShapeDtypeStruct((_NS, _NP), jnp.float32),
        jax.ShapeDtypeStruct((_NP,), jnp.float32),
    ),
    mesh=_mesh,
    compiler_params=pltpu.CompilerParams(needs_layout_passes=False),
    scratch_types=[
        pltpu.VMEM((_NP,), jnp.float32),
        pltpu.VMEM((_NP,), jnp.float32),
        pltpu.VMEM((_C,), jnp.int32),
        pltpu.VMEM((_C,), jnp.int32),
        pltpu.VMEM((_C,), jnp.float32),
        pltpu.VMEM((_NS, _SL), jnp.float32),
        pltpu.VMEM((_SL,), jnp.float32),
        pltpu.VMEM((3, _L), jnp.float32),
        pltpu.SemaphoreType.DMA,
    ],
)
def _sc_fused(u_hbm, src_hbm, dst_hbm, w_hbm, betas_hbm,
              y_hbm, p_hbm, ux_hbm,
              u_v, acc_v, src_v, dst_v, w_v, red_v, sl_v, betas_v, sem):
    sid = lax.axis_index("s")
    base = sid * _C
    nbase = sid * _SL

    cps = [
        pltpu.async_copy(u_hbm, u_v, sem),
        pltpu.async_copy(src_hbm.at[pl.ds(base, _C)], src_v, sem),
        pltpu.async_copy(dst_hbm.at[pl.ds(base, _C)], dst_v, sem),
        pltpu.async_copy(w_hbm.at[pl.ds(base, _C)], w_v, sem),
        pltpu.async_copy(betas_hbm, betas_v, sem),
    ]

    @plsc.parallel_loop(0, _NP // _L, unroll=8)
    def _zero0(i):
        acc_v[pl.ds(i * _L, _L)] = jnp.zeros((_L,), jnp.float32)

    for cp in cps:
        cp.wait()

    for p in range(3):
        @plsc.parallel_loop(0, _C // _L, unroll=16)
        def _edges(i):
            s = src_v[pl.ds(i * _L, _L)]
            d = dst_v[pl.ds(i * _L, _L)]
            wv = w_v[pl.ds(i * _L, _L)]
            vals = plsc.load_gather(u_v, [s]) * wv
            plsc.addupdate_scatter(acc_v, [d], vals)

        pltpu.sync_copy(acc_v, p_hbm.at[sid])
        plsc.subcore_barrier()

        if p < 2:
            @plsc.parallel_loop(0, _NP // _L, unroll=8)
            def _zero(i):
                acc_v[pl.ds(i * _L, _L)] = jnp.zeros((_L,), jnp.float32)

        pltpu.sync_copy(p_hbm.at[:, pl.ds(nbase, _SL)], red_v)
        bvec = betas_v[p, :]
        for k in range(_SL // _L):
            v = red_v[0, pl.ds(k * _L, _L)]
            for j in range(1, _NS):
                v = v + red_v[j, pl.ds(k * _L, _L)]
            sl_v[pl.ds(k * _L, _L)] = v + bvec

        if p < 2:
            pltpu.sync_copy(sl_v, ux_hbm.at[pl.ds(nbase, _SL)])
            plsc.subcore_barrier()
            pltpu.sync_copy(ux_hbm, u_v)
        else:
            @pl.when(sid < _NS - 1)
            def _():
                pltpu.sync_copy(sl_v, y_hbm.at[pl.ds(nbase, _SL)])

            @pl.when(sid == _NS - 1)
            def _():
                pltpu.sync_copy(sl_v.at[pl.ds(0, _N - (_NS - 1) * _SL)],
                                y_hbm.at[pl.ds(nbase, _N - (_NS - 1) * _SL)])


def kernel(x, edge_index, edge_weight, W0, b0, W1, b1, W2, b2):
    u0, betas, src, dst = _prep(x, edge_index, W0, W1, W2, b0, b1, b2)

    y, _, _ = _sc_fused(u0, src, dst, edge_weight, betas)
    return y.reshape(_N, 1)


# speedup vs baseline: 86.9075x; 1.0462x over previous
import functools

import jax
import jax.numpy as jnp
from jax import lax
from jax.experimental import pallas as pl
from jax.experimental.pallas import tpu as pltpu
from jax.experimental.pallas import tpu_sc as plsc

_N = 10000
_NP = 10240
_SL = 640
_E = 160000
_NS = 16
_C = _E // _NS
_L = 16


def _prep_body(x_ref, ei_ref, w0_ref, w1_ref, w2_ref, b0_ref, b1_ref, b2_ref,
               u0_ref, betas_ref, sd_ref):
    hi = lax.Precision.HIGHEST
    w2 = w2_ref[...].reshape(512, 1)
    w12 = jnp.dot(w1_ref[...], w2, precision=hi)
    w012 = jnp.dot(w0_ref[...], w12, precision=hi)
    xv = x_ref[...]
    x_hi = xv.astype(jnp.bfloat16)
    x_lo = (xv - x_hi.astype(jnp.float32)).astype(jnp.bfloat16)
    w_hi = w012.astype(jnp.bfloat16)
    w_lo = (w012 - w_hi.astype(jnp.float32)).astype(jnp.bfloat16)
    f32 = jnp.float32
    u0 = (jnp.dot(x_hi, w_hi, preferred_element_type=f32)
          + jnp.dot(x_lo, w_hi, preferred_element_type=f32)
          + jnp.dot(x_hi, w_lo, preferred_element_type=f32))
    u0_ref[...] = jnp.concatenate(
        [u0[:, 0], jnp.zeros((_NP - _N,), jnp.float32)], axis=0)
    c0 = jnp.dot(b0_ref[...][None, :], w12, precision=hi)
    c1 = jnp.dot(b1_ref[...][None, :], w2, precision=hi)
    betas_ref[...] = jnp.concatenate([
        jnp.broadcast_to(c0, (1, _L)),
        jnp.broadcast_to(c1, (1, _L)),
        jnp.broadcast_to(b2_ref[...][None, :], (1, _L)),
    ], axis=0)
    ei = ei_ref[...]
    sd_ref[...] = jnp.bitwise_or(ei[0], lax.shift_left(ei[1], 16))


_prep = pl.pallas_call(
    _prep_body,
    out_shape=(
        jax.ShapeDtypeStruct((_NP,), jnp.float32),
        jax.ShapeDtypeStruct((3, _L), jnp.float32),
        jax.ShapeDtypeStruct((_E,), jnp.int32),
    ),
)


_mesh = plsc.VectorSubcoreMesh(
    core_axis_name="c", subcore_axis_name="s",
    num_cores=1, num_subcores=_NS)


@functools.partial(
    pl.kernel,
    out_type=(
        jax.ShapeDtypeStruct((_N,), jnp.float32),
        jax.ShapeDtypeStruct((_NS, _NP), jnp.float32),
        jax.ShapeDtypeStruct((_NP,), jnp.float32),
    ),
    mesh=_mesh,
    compiler_params=pltpu.CompilerParams(needs_layout_passes=False),
    scratch_types=[
        pltpu.VMEM((_NP,), jnp.float32),
        pltpu.VMEM((_NP,), jnp.float32),
        pltpu.VMEM((_C,), jnp.int32),
        pltpu.VMEM((_C,), jnp.float32),
        pltpu.VMEM((_NS, _SL), jnp.float32),
        pltpu.VMEM((_SL,), jnp.float32),
        pltpu.VMEM((3, _L), jnp.float32),
        pltpu.SemaphoreType.DMA,
    ],
)
def _sc_fused(u_hbm, sd_hbm, w_hbm, betas_hbm,
              y_hbm, p_hbm, ux_hbm,
              u_v, acc_v, sd_v, w_v, red_v, sl_v, betas_v, sem):
    sid = lax.axis_index("s")
    base = sid * _C
    nbase = sid * _SL

    cps = [
        pltpu.async_copy(u_hbm, u_v, sem),
        pltpu.async_copy(sd_hbm.at[pl.ds(base, _C)], sd_v, sem),
        pltpu.async_copy(w_hbm.at[pl.ds(base, _C)], w_v, sem),
        pltpu.async_copy(betas_hbm, betas_v, sem),
    ]

    @plsc.parallel_loop(0, _NP // _L, unroll=8)
    def _zero0(i):
        acc_v[pl.ds(i * _L, _L)] = jnp.zeros((_L,), jnp.float32)

    for cp in cps:
        cp.wait()

    for p in range(3):
        @plsc.parallel_loop(0, _C // _L, unroll=16)
        def _edges(i):
            sd = sd_v[pl.ds(i * _L, _L)]
            s = jnp.bitwise_and(sd, 0xFFFF)
            d = lax.shift_right_logical(sd, 16)
            wv = w_v[pl.ds(i * _L, _L)]
            vals = plsc.load_gather(u_v, [s]) * wv
            plsc.addupdate_scatter(acc_v, [d], vals)

        pltpu.sync_copy(acc_v, p_hbm.at[sid])
        plsc.subcore_barrier()

        if p < 2:
            @plsc.parallel_loop(0, _NP // _L, unroll=8)
            def _zero(i):
                acc_v[pl.ds(i * _L, _L)] = jnp.zeros((_L,), jnp.float32)

        pltpu.sync_copy(p_hbm.at[:, pl.ds(nbase, _SL)], red_v)
        bvec = betas_v[p, :]
        for k in range(_SL // _L):
            v = red_v[0, pl.ds(k * _L, _L)]
            for j in range(1, _NS):
                v = v + red_v[j, pl.ds(k * _L, _L)]
            sl_v[pl.ds(k * _L, _L)] = v + bvec

        if p < 2:
            pltpu.sync_copy(sl_v, ux_hbm.at[pl.ds(nbase, _SL)])
            plsc.subcore_barrier()
            pltpu.sync_copy(ux_hbm, u_v)
        else:
            @pl.when(sid < _NS - 1)
            def _():
                pltpu.sync_copy(sl_v, y_hbm.at[pl.ds(nbase, _SL)])

            @pl.when(sid == _NS - 1)
            def _():
                pltpu.sync_copy(sl_v.at[pl.ds(0, _N - (_NS - 1) * _SL)],
                                y_hbm.at[pl.ds(nbase, _N - (_NS - 1) * _SL)])


def kernel(x, edge_index, edge_weight, W0, b0, W1, b1, W2, b2):
    u0, betas, sd = _prep(x, edge_index, W0, W1, W2.reshape(512), b0, b1, b2)

    y, _, _ = _sc_fused(u0, sd, edge_weight, betas)
    return y.reshape(_N, 1)


# speedup vs baseline: 90.3168x; 1.0392x over previous
import functools

import jax
import jax.numpy as jnp
from jax import lax
from jax.experimental import pallas as pl
from jax.experimental.pallas import tpu as pltpu
from jax.experimental.pallas import tpu_sc as plsc

_N = 10000
_NP = 10240
_SL = 640
_E = 160000
_NS = 16
_C = _E // _NS
_L = 16


_GB = 5
_XB = 2048
_EB = 32768


def _prep_body(x_ref, ei_ref, w0_ref, w1_ref, w2_ref, b0_ref, b1_ref, b2_ref,
               u0_ref, betas_ref, sd_ref, w012_ref):
    hi = lax.Precision.HIGHEST
    pid = pl.program_id(0)

    @pl.when(pid == 0)
    def _():
        w2 = w2_ref[...].reshape(512, 1)
        w12 = jnp.dot(w1_ref[...], w2, precision=hi)
        w012_ref[...] = jnp.dot(w0_ref[...], w12, precision=hi)
        c0 = jnp.dot(b0_ref[...][None, :], w12, precision=hi)
        c1 = jnp.dot(b1_ref[...][None, :], w2, precision=hi)
        betas_ref[...] = jnp.concatenate([
            jnp.broadcast_to(c0, (1, _L)),
            jnp.broadcast_to(c1, (1, _L)),
            jnp.broadcast_to(b2_ref[...][None, :], (1, _L)),
        ], axis=0)

    w012 = w012_ref[...]
    xv = x_ref[...]
    x_hi = xv.astype(jnp.bfloat16)
    x_lo = (xv - x_hi.astype(jnp.float32)).astype(jnp.bfloat16)
    w_hi = w012.astype(jnp.bfloat16)
    w_lo = (w012 - w_hi.astype(jnp.float32)).astype(jnp.bfloat16)
    f32 = jnp.float32
    u0 = (jnp.dot(x_hi, w_hi, preferred_element_type=f32)
          + jnp.dot(x_lo, w_hi, preferred_element_type=f32)
          + jnp.dot(x_hi, w_lo, preferred_element_type=f32))
    u0_ref[...] = u0[:, 0]
    ei = ei_ref[...]
    sd_ref[...] = jnp.bitwise_or(ei[0], lax.shift_left(ei[1], 16))


_prep = pl.pallas_call(
    _prep_body,
    grid=(_GB,),
    in_specs=[
        pl.BlockSpec((_XB, 256), lambda i: (i, 0)),
        pl.BlockSpec((2, _EB), lambda i: (0, i)),
        pl.BlockSpec((256, 512), lambda i: (0, 0)),
        pl.BlockSpec((512, 512), lambda i: (0, 0)),
        pl.BlockSpec((512,), lambda i: (0,)),
        pl.BlockSpec((512,), lambda i: (0,)),
        pl.BlockSpec((512,), lambda i: (0,)),
        pl.BlockSpec((1,), lambda i: (0,)),
    ],
    out_specs=(
        pl.BlockSpec((_XB,), lambda i: (i,)),
        pl.BlockSpec((3, _L), lambda i: (0, 0)),
        pl.BlockSpec((_EB,), lambda i: (i,)),
    ),
    out_shape=(
        jax.ShapeDtypeStruct((_NP,), jnp.float32),
        jax.ShapeDtypeStruct((3, _L), jnp.float32),
        jax.ShapeDtypeStruct((_E,), jnp.int32),
    ),
    scratch_shapes=[pltpu.VMEM((256, 1), jnp.float32)],
)


_mesh = plsc.VectorSubcoreMesh(
    core_axis_name="c", subcore_axis_name="s",
    num_cores=1, num_subcores=_NS)


@functools.partial(
    pl.kernel,
    out_type=(
        jax.ShapeDtypeStruct((_N,), jnp.float32),
        jax.ShapeDtypeStruct((_NS, _NP), jnp.float32),
        jax.ShapeDtypeStruct((_NP,), jnp.float32),
    ),
    mesh=_mesh,
    compiler_params=pltpu.CompilerParams(needs_layout_passes=False),
    scratch_types=[
        pltpu.VMEM((_NP,), jnp.float32),
        pltpu.VMEM((_NP,), jnp.float32),
        pltpu.VMEM((_C,), jnp.int32),
        pltpu.VMEM((_C,), jnp.float32),
        pltpu.VMEM((_NS, _SL), jnp.float32),
        pltpu.VMEM((_SL,), jnp.float32),
        pltpu.VMEM((3, _L), jnp.float32),
        pltpu.SemaphoreType.DMA,
    ],
)
def _sc_fused(u_hbm, sd_hbm, w_hbm, betas_hbm,
              y_hbm, p_hbm, ux_hbm,
              u_v, acc_v, sd_v, w_v, red_v, sl_v, betas_v, sem):
    sid = lax.axis_index("s")
    base = sid * _C
    nbase = sid * _SL

    cps = [
        pltpu.async_copy(u_hbm, u_v, sem),
        pltpu.async_copy(sd_hbm.at[pl.ds(base, _C)], sd_v, sem),
        pltpu.async_copy(w_hbm.at[pl.ds(base, _C)], w_v, sem),
        pltpu.async_copy(betas_hbm, betas_v, sem),
    ]

    @plsc.parallel_loop(0, _NP // _L, unroll=8)
    def _zero0(i):
        acc_v[pl.ds(i * _L, _L)] = jnp.zeros((_L,), jnp.float32)

    for cp in cps:
        cp.wait()

    for p in range(3):
        @plsc.parallel_loop(0, _C // _L, unroll=16)
        def _edges(i):
            sd = sd_v[pl.ds(i * _L, _L)]
            s = jnp.bitwise_and(sd, 0xFFFF)
            d = lax.shift_right_logical(sd, 16)
            wv = w_v[pl.ds(i * _L, _L)]
            vals = plsc.load_gather(u_v, [s]) * wv
            plsc.addupdate_scatter(acc_v, [d], vals)

        pltpu.sync_copy(acc_v, p_hbm.at[sid])
        plsc.subcore_barrier()

        if p < 2:
            @plsc.parallel_loop(0, _NP // _L, unroll=8)
            def _zero(i):
                acc_v[pl.ds(i * _L, _L)] = jnp.zeros((_L,), jnp.float32)

        pltpu.sync_copy(p_hbm.at[:, pl.ds(nbase, _SL)], red_v)
        bvec = betas_v[p, :]
        for k in range(_SL // _L):
            v = red_v[0, pl.ds(k * _L, _L)]
            for j in range(1, _NS):
                v = v + red_v[j, pl.ds(k * _L, _L)]
            sl_v[pl.ds(k * _L, _L)] = v + bvec

        if p < 2:
            pltpu.sync_copy(sl_v, ux_hbm.at[pl.ds(nbase, _SL)])
            plsc.subcore_barrier()
            pltpu.sync_copy(ux_hbm, u_v)
        else:
            @pl.when(sid < _NS - 1)
            def _():
                pltpu.sync_copy(sl_v, y_hbm.at[pl.ds(nbase, _SL)])

            @pl.when(sid == _NS - 1)
            def _():
                pltpu.sync_copy(sl_v.at[pl.ds(0, _N - (_NS - 1) * _SL)],
                                y_hbm.at[pl.ds(nbase, _N - (_NS - 1) * _SL)])


def kernel(x, edge_index, edge_weight, W0, b0, W1, b1, W2, b2):
    u0, betas, sd = _prep(x, edge_index, W0, W1, W2.reshape(512), b0, b1, b2)

    y, _, _ = _sc_fused(u0, sd, edge_weight, betas)
    return y.reshape(_N, 1)


# speedup vs baseline: 91.3401x; 1.0113x over previous
import functools

import jax
import jax.numpy as jnp
from jax import lax
from jax.experimental import pallas as pl
from jax.experimental.pallas import tpu as pltpu
from jax.experimental.pallas import tpu_sc as plsc

_N = 10000
_NP = 10240
_SL = 640
_E = 160000
_NS = 16
_C = _E // _NS
_L = 16


_GB = 5
_XB = 2048
_EB = 32768


def _prep_body(x_ref, ei_ref, w0_ref, w1_ref, w2_ref, b0_ref, b1_ref, b2_ref,
               u0_ref, betas_ref, sd_ref, w012_ref):
    hi = lax.Precision.HIGHEST
    pid = pl.program_id(0)

    @pl.when(pid == 0)
    def _():
        w2 = w2_ref[...].reshape(512, 1)
        w12 = jnp.dot(w1_ref[...], w2, precision=hi)
        w012_ref[...] = jnp.dot(w0_ref[...], w12, precision=hi)
        c0 = jnp.dot(b0_ref[...][None, :], w12, precision=hi)
        c1 = jnp.dot(b1_ref[...][None, :], w2, precision=hi)
        betas_ref[...] = jnp.concatenate([
            jnp.broadcast_to(c0, (1, _L)),
            jnp.broadcast_to(c1, (1, _L)),
            jnp.broadcast_to(b2_ref[...][None, :], (1, _L)),
        ], axis=0)

    w012 = w012_ref[...]
    xv = x_ref[...]
    x_hi = xv.astype(jnp.bfloat16)
    x_lo = (xv - x_hi.astype(jnp.float32)).astype(jnp.bfloat16)
    w_hi = w012.astype(jnp.bfloat16)
    w_lo = (w012 - w_hi.astype(jnp.float32)).astype(jnp.bfloat16)
    f32 = jnp.float32
    u0 = (jnp.dot(x_hi, w_hi, preferred_element_type=f32)
          + jnp.dot(x_lo, w_hi, preferred_element_type=f32)
          + jnp.dot(x_hi, w_lo, preferred_element_type=f32))
    u0_ref[...] = u0[:, 0]
    ei = ei_ref[...]
    sd_ref[...] = jnp.bitwise_or(ei[0], lax.shift_left(ei[1], 16))


_prep = pl.pallas_call(
    _prep_body,
    grid=(_GB,),
    in_specs=[
        pl.BlockSpec((_XB, 256), lambda i: (i, 0)),
        pl.BlockSpec((2, _EB), lambda i: (0, i)),
        pl.BlockSpec((256, 512), lambda i: (0, 0)),
        pl.BlockSpec((512, 512), lambda i: (0, 0)),
        pl.BlockSpec((512,), lambda i: (0,)),
        pl.BlockSpec((512,), lambda i: (0,)),
        pl.BlockSpec((512,), lambda i: (0,)),
        pl.BlockSpec((1,), lambda i: (0,)),
    ],
    out_specs=(
        pl.BlockSpec((_XB,), lambda i: (i,)),
        pl.BlockSpec((3, _L), lambda i: (0, 0)),
        pl.BlockSpec((_EB,), lambda i: (i,)),
    ),
    out_shape=(
        jax.ShapeDtypeStruct((_NP,), jnp.float32),
        jax.ShapeDtypeStruct((3, _L), jnp.float32),
        jax.ShapeDtypeStruct((_E,), jnp.int32),
    ),
    scratch_shapes=[pltpu.VMEM((256, 1), jnp.float32)],
)


_mesh = plsc.VectorSubcoreMesh(
    core_axis_name="c", subcore_axis_name="s",
    num_cores=1, num_subcores=_NS)


@functools.partial(
    pl.kernel,
    out_type=(
        jax.ShapeDtypeStruct((_N,), jnp.float32),
        jax.ShapeDtypeStruct((_NS, _NP), jnp.float32),
        jax.ShapeDtypeStruct((_NP,), jnp.float32),
    ),
    mesh=_mesh,
    compiler_params=pltpu.CompilerParams(needs_layout_passes=False),
    scratch_types=[
        pltpu.VMEM((_NP,), jnp.float32),
        pltpu.VMEM((_NP,), jnp.float32),
        pltpu.VMEM((_C,), jnp.int32),
        pltpu.VMEM((_C,), jnp.float32),
        pltpu.VMEM((_NS, _SL), jnp.float32),
        pltpu.VMEM((_SL,), jnp.float32),
        pltpu.VMEM((3, _L), jnp.float32),
        pltpu.SemaphoreType.DMA,
    ],
)
def _sc_fused(u_hbm, sd_hbm, w_hbm, betas_hbm,
              y_hbm, p_hbm, ux_hbm,
              u_v, acc_v, sd_v, w_v, red_v, sl_v, betas_v, sem):
    sid = lax.axis_index("s")
    base = sid * _C
    nbase = sid * _SL

    cps = [
        pltpu.async_copy(u_hbm, u_v, sem),
        pltpu.async_copy(sd_hbm.at[pl.ds(base, _C)], sd_v, sem),
        pltpu.async_copy(w_hbm.at[pl.ds(base, _C)], w_v, sem),
        pltpu.async_copy(betas_hbm, betas_v, sem),
    ]

    @plsc.parallel_loop(0, _NP // _L, unroll=8)
    def _zero0(i):
        acc_v[pl.ds(i * _L, _L)] = jnp.zeros((_L,), jnp.float32)

    for cp in cps:
        cp.wait()

    for p in range(3):
        @plsc.parallel_loop(0, _C // _L, unroll=16)
        def _edges(i):
            sd = sd_v[pl.ds(i * _L, _L)]
            s = jnp.bitwise_and(sd, 0xFFFF)
            d = lax.shift_right_logical(sd, 16)
            wv = w_v[pl.ds(i * _L, _L)]
            vals = plsc.load_gather(u_v, [s]) * wv
            plsc.addupdate_scatter(acc_v, [d], vals)

        pltpu.sync_copy(acc_v, p_hbm.at[sid])
        plsc.subcore_barrier()

        red_cp = pltpu.async_copy(p_hbm.at[:, pl.ds(nbase, _SL)], red_v, sem)

        if p < 2:
            @plsc.parallel_loop(0, _NP // _L, unroll=8)
            def _zero(i):
                acc_v[pl.ds(i * _L, _L)] = jnp.zeros((_L,), jnp.float32)

        red_cp.wait()
        bvec = betas_v[p, :]
        for k in range(_SL // _L):
            v = red_v[0, pl.ds(k * _L, _L)]
            for j in range(1, _NS):
                v = v + red_v[j, pl.ds(k * _L, _L)]
            sl_v[pl.ds(k * _L, _L)] = v + bvec

        if p < 2:
            pltpu.sync_copy(sl_v, ux_hbm.at[pl.ds(nbase, _SL)])
            plsc.subcore_barrier()
            pltpu.sync_copy(ux_hbm, u_v)
        else:
            @pl.when(sid < _NS - 1)
            def _():
                pltpu.sync_copy(sl_v, y_hbm.at[pl.ds(nbase, _SL)])

            @pl.when(sid == _NS - 1)
            def _():
                pltpu.sync_copy(sl_v.at[pl.ds(0, _N - (_NS - 1) * _SL)],
                                y_hbm.at[pl.ds(nbase, _N - (_NS - 1) * _SL)])


def kernel(x, edge_index, edge_weight, W0, b0, W1, b1, W2, b2):
    u0, betas, sd = _prep(x, edge_index, W0, W1, W2.reshape(512), b0, b1, b2)

    y, _, _ = _sc_fused(u0, sd, edge_weight, betas)
    return y.reshape(_N, 1)
